# baseline (speedup 1.0000x reference)
import functools

import jax
import jax.numpy as jnp
from jax import lax
from jax.experimental import pallas as pl
from jax.experimental.pallas import tpu as pltpu
from jax.experimental.pallas import tpu_sc as plsc


def _sc_gather(idx, lpe, *, rows_per_w, chunk):
    n_chunks = rows_per_w // chunk
    S = idx.shape[0]
    D = lpe.shape[1]
    mesh = plsc.VectorSubcoreMesh(core_axis_name="c", subcore_axis_name="s")

    @functools.partial(
        pl.kernel,
        out_type=jax.ShapeDtypeStruct((S, D // 2), jnp.uint32),
        mesh=mesh,
        scratch_types=[
            pltpu.VMEM((rows_per_w,), jnp.int32),
            pltpu.VMEM((2, chunk, D), jnp.uint32),
            pltpu.VMEM((2, chunk, D // 2), jnp.uint32),
            pltpu.SemaphoreType.DMA((2,)),
            pltpu.SemaphoreType.DMA((2,)),
        ],
    )
    def k(idx_hbm, lpe_hbm, pe_hbm, idx_all, buf, buf16, sem_g, sem_o):
        wid = lax.axis_index("s") * 2 + lax.axis_index("c")
        base = wid * rows_per_w
        pltpu.sync_copy(idx_hbm.at[pl.ds(base, rows_per_w)], idx_all)

        def gather(c, b):
            pltpu.async_copy(
                lpe_hbm.at[idx_all.at[pl.ds(c * chunk, chunk)]],
                buf.at[b],
                sem_g.at[b],
            )

        def wait_gather(b):
            pltpu.make_async_copy(lpe_hbm.at[pl.ds(0, chunk)], buf.at[b], sem_g.at[b]).wait()

        def wait_out(b):
            pltpu.make_async_copy(buf16.at[b], pe_hbm.at[pl.ds(0, chunk)], sem_o.at[b]).wait()

        gather(0, 0)

        def step(c, carry):
            b = lax.rem(c, 2)
            nxt = c + 1

            @pl.when(c >= 2)
            def _():
                wait_out(b)

            @pl.when(nxt < n_chunks)
            def _():
                gather(nxt, lax.rem(nxt, 2))

            wait_gather(b)

            def pack_row(r, carry2):
                for g in range(D // 32):
                    a = buf[b, r, pl.ds(16 * g, 16)]
                    hi = buf[b, r, pl.ds(D // 2 + 16 * g, 16)]
                    buf16[b, r, pl.ds(16 * g, 16)] = (a >> jnp.uint32(16)) | (
                        hi & jnp.uint32(0xFFFF0000)
                    )
                return carry2

            lax.fori_loop(0, chunk, pack_row, 0)
            pltpu.async_copy(
                buf16.at[b], pe_hbm.at[pl.ds(base + c * chunk, chunk)], sem_o.at[b]
            )
            return carry

        lax.fori_loop(0, n_chunks, step, 0)
        for c_last in range(max(0, n_chunks - 2), n_chunks):
            wait_out(c_last % 2)

    return k(idx, lpe)


def _tc_add(x, pe, *, bs, blk0, prev=None):
    Sk = pe.shape[0]
    S, B, D = x.shape

    def unpack_pe(p):
        lo = lax.bitcast_convert_type(p << jnp.uint32(16), jnp.float32)
        hi = lax.bitcast_convert_type(p & jnp.uint32(0xFFFF0000), jnp.float32)
        return jnp.concatenate([lo, hi], axis=1)

    if prev is None:

        def body(x_ref, pe_ref, o_ref):
            o_ref[...] = x_ref[...] + unpack_pe(pe_ref[...])[:, None, :]

        extra_specs = []
        operands = ()
        aliases = {}
    else:

        def body(prev_ref, x_ref, pe_ref, o_ref):
            del prev_ref
            o_ref[...] = x_ref[...] + unpack_pe(pe_ref[...])[:, None, :]

        extra_specs = [pl.BlockSpec(memory_space=pl.ANY)]
        operands = (prev,)
        aliases = {0: 0}

    return pl.pallas_call(
        body,
        grid=(Sk // bs,),
        in_specs=extra_specs
        + [
            pl.BlockSpec((bs, B, D), lambda i: (i + blk0, 0, 0)),
            pl.BlockSpec((bs, D // 2), lambda i: (i, 0)),
        ],
        out_specs=pl.BlockSpec((bs, B, D), lambda i: (i + blk0, 0, 0)),
        out_shape=jax.ShapeDtypeStruct((S, B, D), jnp.float32),
        input_output_aliases=aliases,
    )(*operands, x, pe)


def _pick_chunk(rows_per_w):
    for c in (32, 24, 16, 8):
        if rows_per_w % c == 0:
            return c
    return rows_per_w


def kernel(x, indices, lpe):
    S, B, D = x.shape
    idx = indices.reshape(S).astype(jnp.int32)
    lpe_u32 = lax.bitcast_convert_type(lpe, jnp.uint32)
    sizes = (1024, 3072)
    bs = 256
    out = None
    s0 = 0
    for sk in sizes:
        idx_k = lax.slice_in_dim(idx, s0, s0 + sk)
        rpw = sk // 32
        pe_k = _sc_gather(idx_k, lpe_u32, rows_per_w=rpw, chunk=_pick_chunk(rpw))
        out = _tc_add(x, pe_k, bs=bs, blk0=s0 // bs, prev=out)
        s0 += sk
    return out


# speedup vs baseline: 1.4381x; 1.4381x over previous
import functools

import jax
import jax.numpy as jnp
from jax import lax
from jax.experimental import pallas as pl
from jax.experimental.pallas import tpu as pltpu
from jax.experimental.pallas import tpu_sc as plsc


def _sc_gather(idx, lpe, *, rows_per_w, chunk):
    n_chunks = rows_per_w // chunk
    S = idx.shape[0]
    D = lpe.shape[1]
    mesh = plsc.VectorSubcoreMesh(core_axis_name="c", subcore_axis_name="s")

    @functools.partial(
        pl.kernel,
        out_type=jax.ShapeDtypeStruct((S, D), jnp.float32),
        mesh=mesh,
        scratch_types=[
            pltpu.VMEM((rows_per_w,), jnp.int32),
            pltpu.VMEM((2, chunk, D), jnp.float32),
            pltpu.SemaphoreType.DMA((2,)),
            pltpu.SemaphoreType.DMA((2,)),
        ],
    )
    def k(idx_hbm, lpe_hbm, pe_hbm, idx_all, buf, sem_g, sem_o):
        wid = lax.axis_index("s") * 2 + lax.axis_index("c")
        base = wid * rows_per_w
        pltpu.sync_copy(idx_hbm.at[pl.ds(base, rows_per_w)], idx_all)

        def gather(c, b):
            pltpu.async_copy(
                lpe_hbm.at[idx_all.at[pl.ds(c * chunk, chunk)]],
                buf.at[b],
                sem_g.at[b],
            )

        def wait_gather(b):
            pltpu.make_async_copy(lpe_hbm.at[pl.ds(0, chunk)], buf.at[b], sem_g.at[b]).wait()

        def wait_out(b):
            pltpu.make_async_copy(buf.at[b], pe_hbm.at[pl.ds(0, chunk)], sem_o.at[b]).wait()

        gather(0, 0)

        def step(c, carry):
            b = lax.rem(c, 2)
            nxt = c + 1

            @pl.when(nxt < n_chunks)
            def _():
                @pl.when(c >= 1)
                def _():
                    wait_out(lax.rem(nxt, 2))

                gather(nxt, lax.rem(nxt, 2))

            wait_gather(b)
            pltpu.async_copy(buf.at[b], pe_hbm.at[pl.ds(base + c * chunk, chunk)], sem_o.at[b])
            return carry

        lax.fori_loop(0, n_chunks, step, 0)
        for c_last in range(max(0, n_chunks - 2), n_chunks):
            wait_out(c_last % 2)

    return k(idx, lpe)


def _tc_add(x, pe, *, bs, blk0, prev=None):
    Sk = pe.shape[0]
    S, B, D = x.shape

    if prev is None:

        def body(x_ref, pe_ref, o_ref):
            o_ref[...] = x_ref[...] + pe_ref[...][:, None, :]

        extra_specs = []
        operands = ()
        aliases = {}
    else:

        def body(prev_ref, x_ref, pe_ref, o_ref):
            del prev_ref
            o_ref[...] = x_ref[...] + pe_ref[...][:, None, :]

        extra_specs = [pl.BlockSpec(memory_space=pl.ANY)]
        operands = (prev,)
        aliases = {0: 0}

    return pl.pallas_call(
        body,
        grid=(Sk // bs,),
        in_specs=extra_specs
        + [
            pl.BlockSpec((bs, B, D), lambda i: (i + blk0, 0, 0)),
            pl.BlockSpec((bs, D), lambda i: (i, 0)),
        ],
        out_specs=pl.BlockSpec((bs, B, D), lambda i: (i + blk0, 0, 0)),
        out_shape=jax.ShapeDtypeStruct((S, B, D), jnp.float32),
        input_output_aliases=aliases,
    )(*operands, x, pe)


def _pick_chunk(rows_per_w):
    for c in (32, 24, 16, 8):
        if rows_per_w % c == 0:
            return c
    return rows_per_w


def kernel(x, indices, lpe):
    S, B, D = x.shape
    idx = indices.reshape(S).astype(jnp.int32)
    sizes = (4096,)
    bs = 256
    out = None
    s0 = 0
    for sk in sizes:
        idx_k = lax.slice_in_dim(idx, s0, s0 + sk)
        rpw = sk // 32
        pe_k = _sc_gather(idx_k, lpe, rows_per_w=rpw, chunk=_pick_chunk(rpw))
        out = _tc_add(x, pe_k, bs=bs, blk0=s0 // bs, prev=out)
        s0 += sk
    return out
